# baseline (speedup 1.0000x reference)
import functools

import jax
import jax.numpy as jnp
from jax import lax
from jax.experimental import pallas as pl
from jax.experimental.pallas import tpu as pltpu
from jax.experimental.pallas import tpu_sc as plsc


def _gather_rows(x2, gidx):
    rows, hdim = x2.shape
    info = plsc.get_sparse_core_info()
    ncores, nsub = info.num_cores, info.num_subcores
    nw = ncores * nsub
    rows_per_w = rows // nw
    chunk = 64
    n_chunks = rows_per_w // chunk
    idx3 = gidx.reshape(nw, n_chunks, chunk)
    mesh = plsc.VectorSubcoreMesh(core_axis_name="c", subcore_axis_name="s")

    @functools.partial(
        pl.kernel,
        mesh=mesh,
        out_type=jax.ShapeDtypeStruct((rows, hdim), jnp.float32),
        scratch_types=[
            pltpu.VMEM((n_chunks, chunk), jnp.int32),
            pltpu.VMEM((chunk, hdim), jnp.float32),
            pltpu.VMEM((chunk, hdim), jnp.float32),
            pltpu.SemaphoreType.DMA,
            pltpu.SemaphoreType.DMA,
            pltpu.SemaphoreType.DMA,
            pltpu.SemaphoreType.DMA,
        ],
    )
    def gk(x_hbm, idx_hbm, out_hbm, idx_v, buf0, buf1, g0, g1, s0, s1):
        wid = lax.axis_index("s") * ncores + lax.axis_index("c")
        bufs, gsems, ssems = (buf0, buf1), (g0, g1), (s0, s1)
        pltpu.sync_copy(idx_hbm.at[wid], idx_v)
        gathers = [None] * n_chunks
        stores = [None] * n_chunks
        for c in range(n_chunks):
            bsel = c % 2
            if c >= 2:
                stores[c - 2].wait()
            gathers[c] = pltpu.async_copy(
                x_hbm.at[idx_v.at[c]], bufs[bsel], gsems[bsel])
            if c >= 1:
                gathers[c - 1].wait()
                stores[c - 1] = pltpu.async_copy(
                    bufs[(c - 1) % 2],
                    out_hbm.at[pl.ds(wid * rows_per_w + (c - 1) * chunk, chunk)],
                    ssems[(c - 1) % 2])
        last = n_chunks - 1
        gathers[last].wait()
        stores[last] = pltpu.async_copy(
            bufs[last % 2],
            out_hbm.at[pl.ds(wid * rows_per_w + last * chunk, chunk)],
            ssems[last % 2])
        stores[last - 1].wait()
        stores[last].wait()

    return gk(x2, idx3)


def _layer(x2, h2, w_self, w_head, bias, mask2):
    rows, hdim = x2.shape
    bm = 256
    grid = (rows // bm,)

    def body(x_ref, h_ref, ws_ref, wh_ref, b_ref, m_ref, o_ref):
        acc = jnp.dot(x_ref[...], ws_ref[...], preferred_element_type=jnp.float32)
        acc = acc + jnp.dot(h_ref[...], wh_ref[...], preferred_element_type=jnp.float32)
        acc = acc + b_ref[...]
        o_ref[...] = jnp.maximum(acc, 0.0) * m_ref[...]

    return pl.pallas_call(
        body,
        grid=grid,
        in_specs=[
            pl.BlockSpec((bm, hdim), lambda i: (i, 0)),
            pl.BlockSpec((bm, hdim), lambda i: (i, 0)),
            pl.BlockSpec((hdim, hdim), lambda i: (0, 0)),
            pl.BlockSpec((hdim, hdim), lambda i: (0, 0)),
            pl.BlockSpec((1, hdim), lambda i: (0, 0)),
            pl.BlockSpec((bm, 1), lambda i: (i, 0)),
        ],
        out_specs=pl.BlockSpec((bm, hdim), lambda i: (i, 0)),
        out_shape=jax.ShapeDtypeStruct((rows, hdim), jnp.float32),
    )(x2, h2, w_self, w_head, bias, mask2)


def kernel(hidden_states, attention_mask, heads, rels, W_self, W_head, b):
    del rels
    bsz, seq, hdim = hidden_states.shape
    rows = bsz * seq
    x2 = hidden_states.reshape(rows, hdim)
    offs = (jnp.arange(bsz, dtype=jnp.int32) * seq)[:, None]
    gidx = (heads.astype(jnp.int32) + offs).reshape(rows)
    mask2 = attention_mask.reshape(rows, 1)
    num_layers = W_self.shape[0]
    for l in range(num_layers):
        h2 = _gather_rows(x2, gidx)
        x2 = _layer(x2, h2, W_self[l], W_head[l], b[l].reshape(1, hdim), mask2)
    return x2.reshape(bsz, seq, hdim)


# speedup vs baseline: 1.0309x; 1.0309x over previous
import functools

import jax
import jax.numpy as jnp
from jax import lax
from jax.experimental import pallas as pl
from jax.experimental.pallas import tpu as pltpu
from jax.experimental.pallas import tpu_sc as plsc


def _gather_rows(x2, gidx):
    rows, hdim = x2.shape
    info = plsc.get_sparse_core_info()
    ncores, nsub = info.num_cores, info.num_subcores
    nw = ncores * nsub
    rows_per_w = rows // nw
    chunk = min(128, rows_per_w)
    n_chunks = rows_per_w // chunk
    mesh = plsc.VectorSubcoreMesh(core_axis_name="c", subcore_axis_name="s")

    @functools.partial(
        pl.kernel,
        mesh=mesh,
        out_type=jax.ShapeDtypeStruct((rows, hdim), jnp.float32),
        scratch_types=[
            pltpu.VMEM((chunk,), jnp.int32),
            pltpu.VMEM((chunk, hdim), jnp.float32),
            pltpu.SemaphoreType.DMA,
        ],
    )
    def gk(x_hbm, idx_hbm, out_hbm, idx_v, rows_v, sem):
        wid = lax.axis_index("s") * ncores + lax.axis_index("c")
        for c in range(n_chunks):
            base = wid * rows_per_w + c * chunk
            pltpu.sync_copy(idx_hbm.at[pl.ds(base, chunk)], idx_v)
            pltpu.async_copy(x_hbm.at[idx_v], rows_v, sem).wait()
            pltpu.sync_copy(rows_v, out_hbm.at[pl.ds(base, chunk)])

    return gk(x2, gidx)


def _layer(x2, h2, w_self_bf, w_head_bf, bias, mask2, layer):
    rows, hdim = x2.shape
    bm = 256
    grid = (rows // bm,)

    def body(x_ref, h_ref, ws_ref, wh_ref, b_ref, m_ref, o_ref):
        xb = x_ref[...].astype(jnp.bfloat16)
        hb = h_ref[...].astype(jnp.bfloat16)
        acc = jnp.dot(xb, ws_ref[0], preferred_element_type=jnp.float32)
        acc = acc + jnp.dot(hb, wh_ref[0], preferred_element_type=jnp.float32)
        acc = acc + b_ref[0]
        o_ref[...] = jnp.maximum(acc, 0.0) * m_ref[...]

    return pl.pallas_call(
        body,
        grid=grid,
        in_specs=[
            pl.BlockSpec((bm, hdim), lambda i: (i, 0)),
            pl.BlockSpec((bm, hdim), lambda i: (i, 0)),
            pl.BlockSpec((1, hdim, hdim), lambda i: (layer, 0, 0)),
            pl.BlockSpec((1, hdim, hdim), lambda i: (layer, 0, 0)),
            pl.BlockSpec((1, 1, hdim), lambda i: (layer, 0, 0)),
            pl.BlockSpec((bm, 1), lambda i: (i, 0)),
        ],
        out_specs=pl.BlockSpec((bm, hdim), lambda i: (i, 0)),
        out_shape=jax.ShapeDtypeStruct((rows, hdim), jnp.float32),
    )(x2, h2, w_self_bf, w_head_bf, bias, mask2)


def kernel(hidden_states, attention_mask, heads, rels, W_self, W_head, b):
    del rels
    bsz, seq, hdim = hidden_states.shape
    rows = bsz * seq
    x2 = hidden_states.reshape(rows, hdim)
    offs = (jnp.arange(bsz, dtype=jnp.int32) * seq)[:, None]
    gidx = (heads.astype(jnp.int32) + offs).reshape(rows)
    mask2 = attention_mask.reshape(rows, 1)
    num_layers = W_self.shape[0]
    ws_bf = W_self.astype(jnp.bfloat16)
    b3 = b.reshape(num_layers, 1, hdim)
    wh_bf = W_head.astype(jnp.bfloat16)
    for l in range(num_layers):
        h2 = _gather_rows(x2, gidx)
        x2 = _layer(x2, h2, ws_bf, wh_bf, b3, mask2, l)
    return x2.reshape(bsz, seq, hdim)


# speedup vs baseline: 1.1496x; 1.1151x over previous
import functools

import jax
import jax.numpy as jnp
from jax import lax
from jax.experimental import pallas as pl
from jax.experimental.pallas import tpu as pltpu
from jax.experimental.pallas import tpu_sc as plsc


def _gather_rows(x2, gidx):
    rows, hdim = x2.shape
    info = plsc.get_sparse_core_info()
    ncores, nsub = info.num_cores, info.num_subcores
    nw = ncores * nsub
    rows_per_w = rows // nw
    chunk = min(256, rows_per_w)
    n_chunks = rows_per_w // chunk
    mesh = plsc.VectorSubcoreMesh(core_axis_name="c", subcore_axis_name="s")

    @functools.partial(
        pl.kernel,
        mesh=mesh,
        out_type=jax.ShapeDtypeStruct((rows, hdim), x2.dtype),
        scratch_types=[
            pltpu.VMEM((chunk,), jnp.int32),
            pltpu.VMEM((chunk, hdim), x2.dtype),
            pltpu.SemaphoreType.DMA,
        ],
    )
    def gk(x_hbm, idx_hbm, out_hbm, idx_v, rows_v, sem):
        wid = lax.axis_index("s") * ncores + lax.axis_index("c")
        for c in range(n_chunks):
            base = wid * rows_per_w + c * chunk
            pltpu.sync_copy(idx_hbm.at[pl.ds(base, chunk)], idx_v)
            pltpu.async_copy(x_hbm.at[idx_v], rows_v, sem).wait()
            pltpu.sync_copy(rows_v, out_hbm.at[pl.ds(base, chunk)])

    return gk(x2, gidx)


def _unpack_halves(p, hdim):
    lo = lax.bitcast_convert_type((p & 0xFFFF).astype(jnp.uint16), jnp.bfloat16)
    hi = lax.bitcast_convert_type(
        lax.shift_right_logical(p, 16).astype(jnp.uint16), jnp.bfloat16)
    return jnp.concatenate([lo, hi], axis=1)


def _pack_halves(y, hdim):
    half = hdim // 2
    lo = lax.bitcast_convert_type(y[:, :half], jnp.uint16).astype(jnp.int32)
    hi = lax.bitcast_convert_type(y[:, half:], jnp.uint16).astype(jnp.int32)
    return lo | lax.shift_left(hi, 16)


def _layer(x2, h2, w_self_bf, w_head_bf, bias, mask2, layer, out_dtype):
    rows = x2.shape[0]
    hdim = w_self_bf.shape[-1]
    bm = 256
    grid = (rows // bm,)

    def body(x_ref, h_ref, ws_ref, wh_ref, b_ref, m_ref, o_ref):
        xb = _unpack_halves(x_ref[...], hdim)
        hb = _unpack_halves(h_ref[...], hdim)
        acc = jnp.dot(xb, ws_ref[0], preferred_element_type=jnp.float32)
        acc = acc + jnp.dot(hb, wh_ref[0], preferred_element_type=jnp.float32)
        acc = acc + b_ref[0]
        y = jnp.maximum(acc, 0.0) * m_ref[...]
        if out_dtype == jnp.float32:
            o_ref[...] = y
        else:
            o_ref[...] = _pack_halves(y.astype(jnp.bfloat16), hdim)

    if out_dtype == jnp.float32:
        out_cols, out_arr_dtype = hdim, jnp.float32
    else:
        out_cols, out_arr_dtype = hdim // 2, jnp.int32
    return pl.pallas_call(
        body,
        grid=grid,
        in_specs=[
            pl.BlockSpec((bm, hdim // 2), lambda i: (i, 0)),
            pl.BlockSpec((bm, hdim // 2), lambda i: (i, 0)),
            pl.BlockSpec((1, hdim, hdim), lambda i: (layer, 0, 0)),
            pl.BlockSpec((1, hdim, hdim), lambda i: (layer, 0, 0)),
            pl.BlockSpec((1, 1, hdim), lambda i: (layer, 0, 0)),
            pl.BlockSpec((bm, 1), lambda i: (i, 0)),
        ],
        out_specs=pl.BlockSpec((bm, out_cols), lambda i: (i, 0)),
        out_shape=jax.ShapeDtypeStruct((rows, out_cols), out_arr_dtype),
    )(x2, h2, w_self_bf, w_head_bf, bias, mask2)


def kernel(hidden_states, attention_mask, heads, rels, W_self, W_head, b):
    del rels
    bsz, seq, hdim = hidden_states.shape
    rows = bsz * seq
    offs = (jnp.arange(bsz, dtype=jnp.int32) * seq)[:, None]
    gidx = (heads.astype(jnp.int32) + offs).reshape(rows)
    mask2 = attention_mask.reshape(rows, 1)
    num_layers = W_self.shape[0]
    ws_bf = W_self.astype(jnp.bfloat16)
    wh_bf = W_head.astype(jnp.bfloat16)
    b3 = b.reshape(num_layers, 1, hdim)
    x0_bf = hidden_states.reshape(rows, hdim).astype(jnp.bfloat16)
    half = hdim // 2
    lo0 = lax.bitcast_convert_type(x0_bf[:, :half], jnp.uint16).astype(jnp.int32)
    hi0 = lax.bitcast_convert_type(x0_bf[:, half:], jnp.uint16).astype(jnp.int32)
    x2 = lo0 | lax.shift_left(hi0, 16)
    for l in range(num_layers):
        out_dtype = jnp.float32 if l == num_layers - 1 else jnp.bfloat16
        h2 = _gather_rows(x2, gidx)
        x2 = _layer(x2, h2, ws_bf, wh_bf, b3, mask2, l, out_dtype)
    return x2.reshape(bsz, seq, hdim)


# speedup vs baseline: 1.1567x; 1.0061x over previous
import functools

import jax
import jax.numpy as jnp
from jax import lax
from jax.experimental import pallas as pl
from jax.experimental.pallas import tpu as pltpu
from jax.experimental.pallas import tpu_sc as plsc


def _gather_rows(x2, heads_flat, seq):
    rows, cols = x2.shape
    info = plsc.get_sparse_core_info()
    ncores, nsub = info.num_cores, info.num_subcores
    nw = ncores * nsub
    rows_per_w = rows // nw
    chunk = rows_per_w
    while chunk * cols + rows_per_w > 120000:
        chunk //= 2
    n_chunks = rows_per_w // chunk
    in_kernel_offset = seq % rows_per_w == 0
    mesh = plsc.VectorSubcoreMesh(core_axis_name="c", subcore_axis_name="s")

    @functools.partial(
        pl.kernel,
        mesh=mesh,
        out_type=jax.ShapeDtypeStruct((rows, cols), x2.dtype),
        scratch_types=[
            pltpu.VMEM((chunk,), jnp.int32),
            pltpu.VMEM((chunk, cols), x2.dtype),
            pltpu.SemaphoreType.DMA,
        ],
    )
    def gk(x_hbm, idx_hbm, out_hbm, idx_v, rows_v, sem):
        wid = lax.axis_index("s") * ncores + lax.axis_index("c")
        for c in range(n_chunks):
            base = wid * rows_per_w + c * chunk
            pltpu.sync_copy(idx_hbm.at[pl.ds(base, chunk)], idx_v)
            if in_kernel_offset:
                off = (base // seq) * seq
                for i in range(chunk // 16):
                    sl = pl.ds(i * 16, 16)
                    idx_v[sl] = idx_v[sl] + off
            pltpu.async_copy(x_hbm.at[idx_v], rows_v, sem).wait()
            pltpu.sync_copy(rows_v, out_hbm.at[pl.ds(base, chunk)])

    if not in_kernel_offset:
        bsz = rows // seq
        offs = (jnp.arange(bsz, dtype=jnp.int32) * seq)[:, None]
        heads_flat = (heads_flat.reshape(bsz, seq) + offs).reshape(rows)
    return gk(x2, heads_flat)


def _unpack_halves(p, hdim):
    lo = lax.bitcast_convert_type((p & 0xFFFF).astype(jnp.uint16), jnp.bfloat16)
    hi = lax.bitcast_convert_type(
        lax.shift_right_logical(p, 16).astype(jnp.uint16), jnp.bfloat16)
    return jnp.concatenate([lo, hi], axis=1)


def _pack_halves(y, hdim):
    half = hdim // 2
    lo = lax.bitcast_convert_type(y[:, :half], jnp.uint16).astype(jnp.int32)
    hi = lax.bitcast_convert_type(y[:, half:], jnp.uint16).astype(jnp.int32)
    return lo | lax.shift_left(hi, 16)


def _layer(x2, h2, w_self_bf, w_head_bf, bias, mask2, layer, in_packed, out_packed):
    rows = x2.shape[0]
    hdim = w_self_bf.shape[-1]
    bm = 256
    grid = (rows // bm,)
    in_cols = hdim // 2 if in_packed else hdim
    out_cols = hdim // 2 if out_packed else hdim
    out_arr_dtype = jnp.int32 if out_packed else jnp.float32

    def body(x_ref, h_ref, ws_ref, wh_ref, b_ref, m_ref, o_ref):
        if in_packed:
            xb = _unpack_halves(x_ref[...], hdim)
            hb = _unpack_halves(h_ref[...], hdim)
        else:
            xb = x_ref[...].astype(jnp.bfloat16)
            hb = h_ref[...].astype(jnp.bfloat16)
        acc = jnp.dot(xb, ws_ref[0], preferred_element_type=jnp.float32)
        acc = acc + jnp.dot(hb, wh_ref[0], preferred_element_type=jnp.float32)
        acc = acc + b_ref[0]
        y = jnp.maximum(acc, 0.0) * m_ref[...]
        if out_packed:
            o_ref[...] = _pack_halves(y.astype(jnp.bfloat16), hdim)
        else:
            o_ref[...] = y

    return pl.pallas_call(
        body,
        grid=grid,
        in_specs=[
            pl.BlockSpec((bm, in_cols), lambda i: (i, 0)),
            pl.BlockSpec((bm, in_cols), lambda i: (i, 0)),
            pl.BlockSpec((1, hdim, hdim), lambda i: (layer, 0, 0)),
            pl.BlockSpec((1, hdim, hdim), lambda i: (layer, 0, 0)),
            pl.BlockSpec((1, 1, hdim), lambda i: (layer, 0, 0)),
            pl.BlockSpec((bm, 1), lambda i: (i, 0)),
        ],
        out_specs=pl.BlockSpec((bm, out_cols), lambda i: (i, 0)),
        out_shape=jax.ShapeDtypeStruct((rows, out_cols), out_arr_dtype),
    )(x2, h2, w_self_bf, w_head_bf, bias, mask2)


def kernel(hidden_states, attention_mask, heads, rels, W_self, W_head, b):
    del rels
    bsz, seq, hdim = hidden_states.shape
    rows = bsz * seq
    heads_flat = heads.astype(jnp.int32).reshape(rows)
    mask2 = attention_mask.reshape(rows, 1)
    num_layers = W_self.shape[0]
    ws_bf = W_self.astype(jnp.bfloat16)
    wh_bf = W_head.astype(jnp.bfloat16)
    b3 = b.reshape(num_layers, 1, hdim)
    x2 = hidden_states.reshape(rows, hdim)
    for l in range(num_layers):
        h2 = _gather_rows(x2, heads_flat, seq)
        x2 = _layer(x2, h2, ws_bf, wh_bf, b3, mask2, l,
                    in_packed=l > 0, out_packed=l < num_layers - 1)
    return x2.reshape(bsz, seq, hdim)
